# Initial kernel scaffold; baseline (speedup 1.0000x reference)
import jax
import jax.numpy as jnp
from jax.experimental import pallas as pl


def kernel(h, x, edges, edge_attr, params):
    raise NotImplementedError("write your pallas kernel here")


# speedup vs baseline: 2.9124x; 2.9124x over previous
import functools

import jax
import jax.numpy as jnp
from jax import lax
from jax.experimental import pallas as pl
from jax.experimental.pallas import tpu as pltpu
from jax.experimental.pallas import tpu_sc as plsc

F32 = jnp.float32
N_NODES = 10000
E_EDGES = 320000
D = 128
ED = 4
DEPTH = 4
CLAMP = 2.0
WX = 16

NC = 2
NS = 16
NW = NC * NS
EW = E_EDGES // NW
K = 80
NITER = EW // K
N_PAD = 10240
ZR = N_PAD // NS
ZB = 128

BE = 2000
BN = 2000

@functools.cache
def _mesh():
    return plsc.VectorSubcoreMesh(core_axis_name="c", subcore_axis_name="s",
                                  num_cores=NC, num_subcores=NS)


def _dot(a, b):
    return jnp.dot(a, b, preferred_element_type=F32)


def _silu(v):
    return v * jax.nn.sigmoid(v)


@functools.cache
def _make_gather2(F):
    @functools.partial(
        pl.kernel,
        out_type=(jax.ShapeDtypeStruct((E_EDGES, F), F32),
                  jax.ShapeDtypeStruct((E_EDGES, F), F32)),
        mesh=_mesh(),
        scratch_types=[
            pltpu.VMEM((NITER, K), jnp.int32),
            pltpu.VMEM((NITER, K), jnp.int32),
            pltpu.VMEM((K, F), F32),
            pltpu.VMEM((K, F), F32),
            pltpu.SemaphoreType.DMA,
            pltpu.SemaphoreType.DMA,
        ],
    )
    def gather2(tA, tB, rowi, coli, outA, outB, rowv, colv, bufA, bufB,
                semA, semB):
        cid = lax.axis_index("c")
        sid = lax.axis_index("s")
        wid = sid * NC + cid
        pltpu.sync_copy(rowi.at[wid], rowv)
        pltpu.sync_copy(coli.at[wid], colv)

        def body(j, carry):
            base = wid * EW + j * K
            cpA = pltpu.async_copy(tA.at[rowv.at[j]], bufA, semA)
            cpB = pltpu.async_copy(tB.at[colv.at[j]], bufB, semB)
            cpA.wait()
            cpB.wait()
            pltpu.sync_copy(bufA, outA.at[pl.ds(base, K)])
            pltpu.sync_copy(bufB, outB.at[pl.ds(base, K)])
            return carry

        lax.fori_loop(0, NITER, body, 0)

    return gather2


def _gather_d(*args):
    return _make_gather2(D)(*args)


def _zero_shared(zbuf, shared, sid, width):
    def zrow(r, carry):
        for c in range(width // 16):
            zbuf[r, pl.ds(c * 16, 16)] = jnp.zeros((16,), F32)
        return carry
    lax.fori_loop(0, ZB, zrow, 0)
    for q in range(ZR // ZB):
        pltpu.sync_copy(zbuf, shared.at[pl.ds(sid * ZR + q * ZB, ZB)])


@functools.cache
def _make_scatter():
    @functools.partial(
        pl.kernel,
        out_type=jax.ShapeDtypeStruct((NC, N_PAD, D), F32),
        mesh=_mesh(),
        scratch_types=[
            pltpu.VMEM((NITER, K), jnp.int32),
            pltpu.VMEM((K, D), F32),
            pltpu.VMEM((ZB, D), F32),
            pltpu.VMEM_SHARED((N_PAD, D), F32),
        ],
    )
    def scatter(vals, rowi, out, rowv, buf, zbuf, agg_sh):
        cid = lax.axis_index("c")
        sid = lax.axis_index("s")
        wid = sid * NC + cid
        _zero_shared(zbuf, agg_sh, sid, D)
        pltpu.sync_copy(rowi.at[wid], rowv)
        plsc.subcore_barrier()

        def body(j, carry):
            base = wid * EW + j * K
            pltpu.sync_copy(vals.at[pl.ds(base, K)], buf)
            pltpu.sync_copy(buf, agg_sh.at[rowv.at[j]], add=True)
            return carry

        lax.fori_loop(0, NITER, body, 0)
        plsc.subcore_barrier()
        pltpu.sync_copy(agg_sh.at[pl.ds(sid * ZR, ZR)],
                        out.at[cid, pl.ds(sid * ZR, ZR)])

    return scatter


def _scatter_sum(*args):
    return _make_scatter()(*args)


def _full(shape):
    return pl.BlockSpec(shape, lambda i: (0,) * len(shape))


def _rows(block):
    return pl.BlockSpec(block, lambda i: (i,) + (0,) * (len(block) - 1))


def _edge_prep_body(xr, xc, ea, diff_o, eaq_o):
    d = xr[...] - xc[...]
    diff_o[...] = d[:, :WX]
    radial = jnp.sum(d * d, axis=1, keepdims=True)
    eaq_o[...] = jnp.concatenate(
        [radial, ea[...], jnp.zeros((radial.shape[0], 3), F32)], axis=1)


def _edge_prep(xr, xc, ea):
    return pl.pallas_call(
        _edge_prep_body,
        grid=(E_EDGES // BE,),
        in_specs=[_rows((BE, D)), _rows((BE, D)), _rows((BE, ED))],
        out_specs=[_rows((BE, WX)), _rows((BE, 8))],
        out_shape=[jax.ShapeDtypeStruct((E_EDGES, WX), F32),
                   jax.ShapeDtypeStruct((E_EDGES, 8), F32)],
    )(xr, xc, ea)


def _m1(a, b, q, w384, b1):
    e_in = jnp.concatenate(
        [a[...], b[...], q[...], jnp.zeros((BE, 384 - 2 * D - 8), F32)],
        axis=1)
    return _silu(_dot(e_in, w384[...]) + b1[...])


def _edge_mid_body(a, b, q, w384, b1, w2, b2, m2_o):
    m1 = _m1(a, b, q, w384, b1)
    m2_o[...] = _silu(_dot(m1, w2[...]) + b2[...])


def _edge_mid(A, B, eaq, w384, b1, w2, b2):
    return pl.pallas_call(
        _edge_mid_body,
        grid=(E_EDGES // BE,),
        in_specs=[_rows((BE, D)), _rows((BE, D)), _rows((BE, 8)),
                  _full((384, D)), _full((1, D)), _full((D, D)), _full((1, D))],
        out_specs=_rows((BE, D)),
        out_shape=jax.ShapeDtypeStruct((E_EDGES, D), F32),
    )(A, B, eaq, w384, b1, w2, b2)


def _edge_last_body(a, b, q, diff, w384, b1, w2, b2, wc1, bc1, wc2, m2_o, wc_o):
    m1 = _m1(a, b, q, w384, b1)
    m2 = _silu(_dot(m1, w2[...]) + b2[...])
    m2_o[...] = m2
    t = _silu(_dot(m2, wc1[...]) + bc1[...])
    w = _dot(t, wc2[...])[:, 0:1]
    w = jnp.clip(w, -CLAMP, CLAMP)
    lane = lax.broadcasted_iota(jnp.int32, (BE, WX), 1)
    wc16 = jnp.where(lane == 3, 1.0, diff[...] * w)
    wc_o[...] = jnp.concatenate(
        [wc16, jnp.zeros((BE, D - WX), F32)], axis=1)


def _edge_last(A, B, eaq, diff, w384, b1, w2, b2, wc1, bc1, wc2):
    return pl.pallas_call(
        _edge_last_body,
        grid=(E_EDGES // BE,),
        in_specs=[_rows((BE, D)), _rows((BE, D)), _rows((BE, 8)),
                  _rows((BE, WX)), _full((384, D)), _full((1, D)),
                  _full((D, D)), _full((1, D)), _full((D, D)),
                  _full((1, D)), _full((D, 8))],
        out_specs=[_rows((BE, D)), _rows((BE, D))],
        out_shape=[jax.ShapeDtypeStruct((E_EDGES, D), F32),
                   jax.ShapeDtypeStruct((E_EDGES, D), F32)],
    )(A, B, eaq, diff, w384, b1, w2, b2, wc1, bc1, wc2)


def _node_mid_body(h, g0, g1, wn1, bn1, wn2, bn2, hn_o):
    hv = h[...]
    agg = g0[...] + g1[...]
    cat = jnp.concatenate([hv, agg], axis=1)
    t = _silu(_dot(cat, wn1[...]) + bn1[...])
    hn_o[...] = hv + _dot(t, wn2[...]) + bn2[...]


def _node_mid(h, g0, g1, wn1, bn1, wn2, bn2):
    return pl.pallas_call(
        _node_mid_body,
        grid=(N_NODES // BN,),
        in_specs=[_rows((BN, D)), _rows((BN, D)), _rows((BN, D)),
                  _full((2 * D, D)), _full((1, D)),
                  _full((D, D)), _full((1, D))],
        out_specs=_rows((BN, D)),
        out_shape=jax.ShapeDtypeStruct((N_NODES, D), F32),
    )(h, g0, g1, wn1, bn1, wn2, bn2)


def _coord_body(x, s0, s1, x_o):
    s = s0[...] + s1[...]
    cnt = jnp.clip(s[:, 3:4], 1.0, None)
    x_o[...] = x[...] + s[:, 0:3] / cnt


def _coord_final(x, sx0, sx1):
    return pl.pallas_call(
        _coord_body,
        grid=(N_NODES // BN,),
        in_specs=[_rows((BN, 3)), _rows((BN, D)), _rows((BN, D))],
        out_specs=_rows((BN, 3)),
        out_shape=jax.ShapeDtypeStruct((N_NODES, 3), F32),
    )(x, sx0, sx1)


def kernel(h, x, edges, edge_attr, params):
    row = edges[0].astype(jnp.int32)
    col = edges[1].astype(jnp.int32)
    rowi = row.reshape(NW, NITER, K)
    coli = col.reshape(NW, NITER, K)

    xpad = jnp.pad(x, ((0, 0), (0, D - 3)))
    xr, xc = _gather_d(xpad, xpad, rowi, coli)
    diff, eaq = _edge_prep(xr, xc, edge_attr)

    for l in range(DEPTH):
        p = params[l]
        w384 = jnp.pad(p['We1'], ((0, 384 - 2 * D - 1 - ED), (0, 0)))
        b1 = p['be1'].reshape(1, D)
        b2 = p['be2'].reshape(1, D)
        bn1 = p['bn1'].reshape(1, D)
        bn2 = p['bn2'].reshape(1, D)
        A, B = _gather_d(h, h, rowi, coli)
        if l < DEPTH - 1:
            m2 = _edge_mid(A, B, eaq, w384, b1, p['We2'], b2)
            agg = _scatter_sum(m2, rowi)
            h = _node_mid(h, agg[0, :N_NODES], agg[1, :N_NODES],
                          p['Wn1'], bn1, p['Wn2'], bn2)
        else:
            wc2 = jnp.pad(p['Wc2'], ((0, 0), (0, 7)))
            m2, wcv = _edge_last(A, B, eaq, diff, w384, b1, p['We2'], b2,
                                 p['Wc1'], p['bc1'].reshape(1, D), wc2)
            agg = _scatter_sum(m2, rowi)
            aggx = _scatter_sum(wcv, rowi)
            h = _node_mid(h, agg[0, :N_NODES], agg[1, :N_NODES],
                          p['Wn1'], bn1, p['Wn2'], bn2)
            x = _coord_final(x, aggx[0, :N_NODES], aggx[1, :N_NODES])

    return h, x, edge_attr


# speedup vs baseline: 3.0035x; 1.0313x over previous
import functools

import jax
import jax.numpy as jnp
from jax import lax
from jax.experimental import pallas as pl
from jax.experimental.pallas import tpu as pltpu
from jax.experimental.pallas import tpu_sc as plsc

F32 = jnp.float32
N_NODES = 10000
E_EDGES = 320000
D = 128
ED = 4
DEPTH = 4
CLAMP = 2.0
WX = 16

NC = 2
NS = 16
NW = NC * NS
EW = E_EDGES // NW
K = 80
NITER = EW // K
KG = 50
QG = 8
NOG = EW // (KG * QG)
N_PAD = 10240
ZR = N_PAD // NS
ZB = 128

BE = 2000
BN = 2000

@functools.cache
def _mesh():
    return plsc.VectorSubcoreMesh(core_axis_name="c", subcore_axis_name="s",
                                  num_cores=NC, num_subcores=NS)


def _dot(a, b):
    return jnp.dot(a, b, preferred_element_type=F32)


def _silu(v):
    return v * jax.nn.sigmoid(v)


@functools.cache
def _make_gather2(F):
    @functools.partial(
        pl.kernel,
        out_type=(jax.ShapeDtypeStruct((E_EDGES, F), F32),
                  jax.ShapeDtypeStruct((E_EDGES, F), F32)),
        mesh=_mesh(),
        scratch_types=[
            pltpu.VMEM((QG, KG), jnp.int32),
            pltpu.VMEM((QG, KG), jnp.int32),
            pltpu.VMEM((QG * KG, F), F32),
            pltpu.VMEM((QG * KG, F), F32),
            pltpu.SemaphoreType.DMA,
            pltpu.SemaphoreType.DMA,
        ],
    )
    def gather2(tA, tB, rowi, coli, outA, outB, rowv, colv, bufA, bufB,
                semA, semB):
        cid = lax.axis_index("c")
        sid = lax.axis_index("s")
        wid = sid * NC + cid

        def body(jo, carry):
            pltpu.sync_copy(rowi.at[wid].at[pl.ds(jo * QG, QG)], rowv)
            pltpu.sync_copy(coli.at[wid].at[pl.ds(jo * QG, QG)], colv)
            cps = []
            for q in range(QG):
                cps.append(pltpu.async_copy(
                    tA.at[rowv.at[q]], bufA.at[pl.ds(q * KG, KG)], semA))
                cps.append(pltpu.async_copy(
                    tB.at[colv.at[q]], bufB.at[pl.ds(q * KG, KG)], semB))
            for cp in cps:
                cp.wait()
            base = wid * EW + jo * (QG * KG)
            pltpu.sync_copy(bufA, outA.at[pl.ds(base, QG * KG)])
            pltpu.sync_copy(bufB, outB.at[pl.ds(base, QG * KG)])
            return carry

        lax.fori_loop(0, NOG, body, 0)

    return gather2


def _gather_d(*args):
    return _make_gather2(D)(*args)


def _zero_shared(zbuf, shared, sid, width):
    def zrow(r, carry):
        for c in range(width // 16):
            zbuf[r, pl.ds(c * 16, 16)] = jnp.zeros((16,), F32)
        return carry
    lax.fori_loop(0, ZB, zrow, 0)
    for q in range(ZR // ZB):
        pltpu.sync_copy(zbuf, shared.at[pl.ds(sid * ZR + q * ZB, ZB)])


@functools.cache
def _make_scatter():
    @functools.partial(
        pl.kernel,
        out_type=jax.ShapeDtypeStruct((NC, N_PAD, D), F32),
        mesh=_mesh(),
        scratch_types=[
            pltpu.VMEM((NITER, K), jnp.int32),
            pltpu.VMEM((K, D), F32),
            pltpu.VMEM((ZB, D), F32),
            pltpu.VMEM_SHARED((N_PAD, D), F32),
        ],
    )
    def scatter(vals, rowi, out, rowv, buf, zbuf, agg_sh):
        cid = lax.axis_index("c")
        sid = lax.axis_index("s")
        wid = sid * NC + cid
        _zero_shared(zbuf, agg_sh, sid, D)
        pltpu.sync_copy(rowi.at[wid], rowv)
        plsc.subcore_barrier()

        def body(j, carry):
            base = wid * EW + j * K
            pltpu.sync_copy(vals.at[pl.ds(base, K)], buf)
            pltpu.sync_copy(buf, agg_sh.at[rowv.at[j]], add=True)
            return carry

        lax.fori_loop(0, NITER, body, 0)
        plsc.subcore_barrier()
        pltpu.sync_copy(agg_sh.at[pl.ds(sid * ZR, ZR)],
                        out.at[cid, pl.ds(sid * ZR, ZR)])

    return scatter


def _scatter_sum(*args):
    return _make_scatter()(*args)


def _full(shape):
    return pl.BlockSpec(shape, lambda i: (0,) * len(shape))


def _rows(block):
    return pl.BlockSpec(block, lambda i: (i,) + (0,) * (len(block) - 1))


def _edge_prep_body(xr, xc, ea, diff_o, eaq_o):
    d = xr[...] - xc[...]
    diff_o[...] = d[:, :WX]
    radial = jnp.sum(d * d, axis=1, keepdims=True)
    eaq_o[...] = jnp.concatenate(
        [radial, ea[...], jnp.zeros((radial.shape[0], 3), F32)], axis=1)


def _edge_prep(xr, xc, ea):
    return pl.pallas_call(
        _edge_prep_body,
        grid=(E_EDGES // BE,),
        in_specs=[_rows((BE, D)), _rows((BE, D)), _rows((BE, ED))],
        out_specs=[_rows((BE, WX)), _rows((BE, 8))],
        out_shape=[jax.ShapeDtypeStruct((E_EDGES, WX), F32),
                   jax.ShapeDtypeStruct((E_EDGES, 8), F32)],
    )(xr, xc, ea)


def _m1(a, b, q, w384, b1):
    e_in = jnp.concatenate(
        [a[...], b[...], q[...], jnp.zeros((BE, 384 - 2 * D - 8), F32)],
        axis=1)
    return _silu(_dot(e_in, w384[...]) + b1[...])


def _edge_mid_body(a, b, q, w384, b1, w2, b2, m2_o):
    m1 = _m1(a, b, q, w384, b1)
    m2_o[...] = _silu(_dot(m1, w2[...]) + b2[...])


def _edge_mid(A, B, eaq, w384, b1, w2, b2):
    return pl.pallas_call(
        _edge_mid_body,
        grid=(E_EDGES // BE,),
        in_specs=[_rows((BE, D)), _rows((BE, D)), _rows((BE, 8)),
                  _full((384, D)), _full((1, D)), _full((D, D)), _full((1, D))],
        out_specs=_rows((BE, D)),
        out_shape=jax.ShapeDtypeStruct((E_EDGES, D), F32),
    )(A, B, eaq, w384, b1, w2, b2)


def _edge_last_body(a, b, q, diff, w384, b1, w2, b2, wc1, bc1, wc2, m2_o, wc_o):
    m1 = _m1(a, b, q, w384, b1)
    m2 = _silu(_dot(m1, w2[...]) + b2[...])
    m2_o[...] = m2
    t = _silu(_dot(m2, wc1[...]) + bc1[...])
    w = _dot(t, wc2[...])[:, 0:1]
    w = jnp.clip(w, -CLAMP, CLAMP)
    lane = lax.broadcasted_iota(jnp.int32, (BE, WX), 1)
    wc16 = jnp.where(lane == 3, 1.0, diff[...] * w)
    wc_o[...] = jnp.concatenate(
        [wc16, jnp.zeros((BE, D - WX), F32)], axis=1)


def _edge_last(A, B, eaq, diff, w384, b1, w2, b2, wc1, bc1, wc2):
    return pl.pallas_call(
        _edge_last_body,
        grid=(E_EDGES // BE,),
        in_specs=[_rows((BE, D)), _rows((BE, D)), _rows((BE, 8)),
                  _rows((BE, WX)), _full((384, D)), _full((1, D)),
                  _full((D, D)), _full((1, D)), _full((D, D)),
                  _full((1, D)), _full((D, 8))],
        out_specs=[_rows((BE, D)), _rows((BE, D))],
        out_shape=[jax.ShapeDtypeStruct((E_EDGES, D), F32),
                   jax.ShapeDtypeStruct((E_EDGES, D), F32)],
    )(A, B, eaq, diff, w384, b1, w2, b2, wc1, bc1, wc2)


def _node_mid_body(h, g0, g1, wn1, bn1, wn2, bn2, hn_o):
    hv = h[...]
    agg = g0[...] + g1[...]
    cat = jnp.concatenate([hv, agg], axis=1)
    t = _silu(_dot(cat, wn1[...]) + bn1[...])
    hn_o[...] = hv + _dot(t, wn2[...]) + bn2[...]


def _node_mid(h, g0, g1, wn1, bn1, wn2, bn2):
    return pl.pallas_call(
        _node_mid_body,
        grid=(N_NODES // BN,),
        in_specs=[_rows((BN, D)), _rows((BN, D)), _rows((BN, D)),
                  _full((2 * D, D)), _full((1, D)),
                  _full((D, D)), _full((1, D))],
        out_specs=_rows((BN, D)),
        out_shape=jax.ShapeDtypeStruct((N_NODES, D), F32),
    )(h, g0, g1, wn1, bn1, wn2, bn2)


def _coord_body(x, s0, s1, x_o):
    s = s0[...] + s1[...]
    cnt = jnp.clip(s[:, 3:4], 1.0, None)
    x_o[...] = x[...] + s[:, 0:3] / cnt


def _coord_final(x, sx0, sx1):
    return pl.pallas_call(
        _coord_body,
        grid=(N_NODES // BN,),
        in_specs=[_rows((BN, 3)), _rows((BN, D)), _rows((BN, D))],
        out_specs=_rows((BN, 3)),
        out_shape=jax.ShapeDtypeStruct((N_NODES, 3), F32),
    )(x, sx0, sx1)


def kernel(h, x, edges, edge_attr, params):
    row = edges[0].astype(jnp.int32)
    col = edges[1].astype(jnp.int32)
    rowi = row.reshape(NW, NITER, K)
    rowg = row.reshape(NW, EW // KG, KG)
    colg = col.reshape(NW, EW // KG, KG)

    xpad = jnp.pad(x, ((0, 0), (0, D - 3)))
    xr, xc = _gather_d(xpad, xpad, rowg, colg)
    diff, eaq = _edge_prep(xr, xc, edge_attr)

    for l in range(DEPTH):
        p = params[l]
        w384 = jnp.pad(p['We1'], ((0, 384 - 2 * D - 1 - ED), (0, 0)))
        b1 = p['be1'].reshape(1, D)
        b2 = p['be2'].reshape(1, D)
        bn1 = p['bn1'].reshape(1, D)
        bn2 = p['bn2'].reshape(1, D)
        A, B = _gather_d(h, h, rowg, colg)
        if l < DEPTH - 1:
            m2 = _edge_mid(A, B, eaq, w384, b1, p['We2'], b2)
            agg = _scatter_sum(m2, rowi)
            h = _node_mid(h, agg[0, :N_NODES], agg[1, :N_NODES],
                          p['Wn1'], bn1, p['Wn2'], bn2)
        else:
            wc2 = jnp.pad(p['Wc2'], ((0, 0), (0, 7)))
            m2, wcv = _edge_last(A, B, eaq, diff, w384, b1, p['We2'], b2,
                                 p['Wc1'], p['bc1'].reshape(1, D), wc2)
            agg = _scatter_sum(m2, rowi)
            aggx = _scatter_sum(wcv, rowi)
            h = _node_mid(h, agg[0, :N_NODES], agg[1, :N_NODES],
                          p['Wn1'], bn1, p['Wn2'], bn2)
            x = _coord_final(x, aggx[0, :N_NODES], aggx[1, :N_NODES])

    return h, x, edge_attr
